# Initial kernel scaffold; baseline (speedup 1.0000x reference)
import jax
import jax.numpy as jnp
from jax.experimental import pallas as pl


def kernel(data, idx):
    raise NotImplementedError("write your pallas kernel here")


# speedup vs baseline: 1.4192x; 1.4192x over previous
import functools

import jax
import jax.numpy as jnp
from jax import lax
from jax.experimental import pallas as pl
from jax.experimental.pallas import tpu as pltpu
from jax.experimental.pallas import tpu_sc as plsc

V, F = 8192, 1024
N = 4096

NC, NS = 2, 16
NW = NC * NS
B_PER_W = N // NW
CHUNK = 32
NCHUNK = B_PER_W // CHUNK

_mesh = plsc.VectorSubcoreMesh(core_axis_name="c", subcore_axis_name="s")


@functools.partial(
    pl.kernel,
    out_type=jax.ShapeDtypeStruct((N, F), jnp.float32),
    mesh=_mesh,
    scratch_types=[
        pltpu.VMEM((B_PER_W,), jnp.int32),
        pltpu.VMEM((2, CHUNK, F), jnp.float32),
        pltpu.SemaphoreType.DMA,
        pltpu.SemaphoreType.DMA,
    ],
)
def _gather_rows(data_hbm, idx_hbm, out_hbm, idx_v, buf_v, gsem, ssem):
    wid = lax.axis_index("s") * NC + lax.axis_index("c")
    base = wid * B_PER_W
    pltpu.sync_copy(idx_hbm.at[pl.ds(base, B_PER_W)], idx_v)

    gathers = [None] * NCHUNK
    stores = [None] * NCHUNK
    gathers[0] = pltpu.async_copy(
        data_hbm.at[idx_v.at[pl.ds(0, CHUNK)]], buf_v.at[0], gsem)
    for c in range(NCHUNK):
        nxt = c + 1
        if nxt < NCHUNK:
            if nxt >= 2:
                stores[nxt - 2].wait()
            gathers[nxt] = pltpu.async_copy(
                data_hbm.at[idx_v.at[pl.ds(nxt * CHUNK, CHUNK)]],
                buf_v.at[nxt % 2], gsem)
        gathers[c].wait()
        stores[c] = pltpu.async_copy(
            buf_v.at[c % 2], out_hbm.at[pl.ds(base + c * CHUNK, CHUNK)], ssem)
    stores[NCHUNK - 2].wait()
    stores[NCHUNK - 1].wait()


def kernel(data, idx):
    return _gather_rows(data, idx[:, 0])
